# baseline (speedup 1.0000x reference)
import functools

import jax
import jax.numpy as jnp
from jax import lax
from jax.experimental import pallas as pl
from jax.experimental.pallas import tpu as pltpu
from jax.experimental.pallas import tpu_sc as plsc

N = 10000
D = 128
E = 320000
NC = 2
NS = 16
NW = NC * NS
K = 128
NCHUNK = 80
E_PAD = NW * NCHUNK * K
N_PAD = 10112
ROWS_PER_TILE = N_PAD // NS


def _sc_aggregate(x, src3d, dst3d, zeros):
    mesh = plsc.VectorSubcoreMesh(core_axis_name="c", subcore_axis_name="s")

    @functools.partial(
        pl.kernel,
        mesh=mesh,
        out_type=jax.ShapeDtypeStruct((NC, N_PAD, D), jnp.float32),
        scratch_types=[
            pltpu.VMEM((NCHUNK, K), jnp.int32),
            pltpu.VMEM((NCHUNK, K), jnp.int32),
            pltpu.VMEM((K, D), jnp.float32),
            pltpu.VMEM_SHARED((N_PAD, D), jnp.float32),
            pltpu.SemaphoreType.DMA,
        ],
    )
    def k(x_hbm, src_hbm, dst_hbm, zeros_hbm, out_hbm, src_v, dst_v, buf, acc, sem):
        c = lax.axis_index("c")
        s = lax.axis_index("s")
        wid = c * NS + s
        pltpu.sync_copy(src_hbm.at[wid], src_v)
        pltpu.sync_copy(dst_hbm.at[wid], dst_v)
        pltpu.sync_copy(
            zeros_hbm.at[pl.ds(s * ROWS_PER_TILE, ROWS_PER_TILE)],
            acc.at[pl.ds(s * ROWS_PER_TILE, ROWS_PER_TILE)],
        )
        plsc.subcore_barrier()

        def body(j, carry):
            pltpu.async_copy(x_hbm.at[src_v.at[j]], buf, sem).wait()
            pltpu.sync_copy(buf, acc.at[dst_v.at[j]], add=True)
            return carry

        lax.fori_loop(0, NCHUNK, body, 0)
        plsc.subcore_barrier()
        pltpu.sync_copy(
            acc.at[pl.ds(s * ROWS_PER_TILE, ROWS_PER_TILE)],
            out_hbm.at[c, pl.ds(s * ROWS_PER_TILE, ROWS_PER_TILE)],
        )

    return k(x, src3d, dst3d, zeros)


def _tc_body(p_ref, x_ref, wlt_ref, bl_ref, wrt_ref, gnw_ref, gnb_ref,
             gnms_ref, w1t_ref, b1_ref, w2t_ref, b2_ref, o_ref):
    agg = p_ref[0, :N, :] + p_ref[1, :N, :]
    h = (jnp.dot(agg, wlt_ref[...], preferred_element_type=jnp.float32)
         + bl_ref[...]
         + jnp.dot(x_ref[...], wrt_ref[...], preferred_element_type=jnp.float32))
    mean = jnp.mean(h, axis=0, keepdims=True)
    cen = h - gnms_ref[...] * mean
    var = jnp.mean(cen * cen, axis=0, keepdims=True)
    inv = lax.rsqrt(var + 1e-5)
    h = gnw_ref[...] * cen * inv + gnb_ref[...]
    h = jnp.maximum(h, 0.0)
    h = jnp.maximum(
        jnp.dot(h, w1t_ref[...], preferred_element_type=jnp.float32) + b1_ref[...],
        0.0)
    o_ref[...] = (jnp.dot(h, w2t_ref[...], preferred_element_type=jnp.float32)
                  + b2_ref[...])


def kernel(x, edge_index, Wl, bl, Wr, gn_w, gn_b, gn_ms, W1, b1, W2, b2):
    pad = E_PAD - E
    src3d = jnp.concatenate(
        [edge_index[0], jnp.zeros((pad,), jnp.int32)]).reshape(NW, NCHUNK, K)
    pad_dst = N + (jnp.arange(pad, dtype=jnp.int32) % (N_PAD - N))
    dst3d = jnp.concatenate([edge_index[1], pad_dst]).reshape(NW, NCHUNK, K)
    zeros = jnp.zeros((N_PAD, D), dtype=jnp.float32)
    partial = _sc_aggregate(x, src3d, dst3d, zeros)

    return pl.pallas_call(
        _tc_body,
        out_shape=jax.ShapeDtypeStruct((N, D), jnp.float32),
    )(partial, x, Wl.T, bl.reshape(1, D), Wr.T,
      gn_w.reshape(1, D), gn_b.reshape(1, D), gn_ms.reshape(1, D),
      W1.T, b1.reshape(1, D), W2.T, b2.reshape(1, D))


# speedup vs baseline: 1.6966x; 1.6966x over previous
import functools

import jax
import jax.numpy as jnp
from jax import lax
from jax.experimental import pallas as pl
from jax.experimental.pallas import tpu as pltpu
from jax.experimental.pallas import tpu_sc as plsc

N = 10000
D = 128
E = 320000
NC = 2
NS = 16
NW = NC * NS
K = 120
NCHUNK = 84
E_PAD = NW * NCHUNK * K
N_PAD = 10112
ROWS_PER_TILE = N_PAD // NS


def _sc_aggregate(x, src3d, dst3d, zeros):
    mesh = plsc.VectorSubcoreMesh(core_axis_name="c", subcore_axis_name="s")

    @functools.partial(
        pl.kernel,
        mesh=mesh,
        out_type=jax.ShapeDtypeStruct((NC, N_PAD, D), jnp.float32),
        scratch_types=[
            pltpu.VMEM((NCHUNK, K), jnp.int32),
            pltpu.VMEM((NCHUNK, K), jnp.int32),
            pltpu.VMEM((K, D), jnp.float32),
            pltpu.VMEM_SHARED((N_PAD, D), jnp.float32),
            pltpu.SemaphoreType.DMA,
        ],
    )
    def k(x_hbm, src_hbm, dst_hbm, zeros_hbm, out_hbm, src_v, dst_v, buf, acc, sem):
        c = lax.axis_index("c")
        s = lax.axis_index("s")
        wid = c * NS + s
        pltpu.sync_copy(src_hbm.at[wid], src_v)
        pltpu.sync_copy(dst_hbm.at[wid], dst_v)
        pltpu.sync_copy(
            zeros_hbm.at[pl.ds(s * ROWS_PER_TILE, ROWS_PER_TILE)],
            acc.at[pl.ds(s * ROWS_PER_TILE, ROWS_PER_TILE)],
        )
        plsc.subcore_barrier()

        def body(j, carry):
            pltpu.async_copy(x_hbm.at[src_v.at[j]], buf, sem).wait()
            pltpu.sync_copy(buf, acc.at[dst_v.at[j]], add=True)
            return carry

        lax.fori_loop(0, NCHUNK, body, 0)
        plsc.subcore_barrier()
        pltpu.sync_copy(
            acc.at[pl.ds(s * ROWS_PER_TILE, ROWS_PER_TILE)],
            out_hbm.at[c, pl.ds(s * ROWS_PER_TILE, ROWS_PER_TILE)],
        )

    return k(x, src3d, dst3d, zeros)


def _tc_body(p_ref, x_ref, wlt_ref, bl_ref, wrt_ref, gnw_ref, gnb_ref,
             gnms_ref, w1t_ref, b1_ref, w2t_ref, b2_ref, o_ref):
    agg = p_ref[0, :N, :] + p_ref[1, :N, :]
    h = (jnp.dot(agg, wlt_ref[...], preferred_element_type=jnp.float32)
         + bl_ref[...]
         + jnp.dot(x_ref[...], wrt_ref[...], preferred_element_type=jnp.float32))
    mean = jnp.mean(h, axis=0, keepdims=True)
    cen = h - gnms_ref[...] * mean
    var = jnp.mean(cen * cen, axis=0, keepdims=True)
    inv = lax.rsqrt(var + 1e-5)
    h = gnw_ref[...] * cen * inv + gnb_ref[...]
    h = jnp.maximum(h, 0.0)
    h = jnp.maximum(
        jnp.dot(h, w1t_ref[...], preferred_element_type=jnp.float32) + b1_ref[...],
        0.0)
    o_ref[...] = (jnp.dot(h, w2t_ref[...], preferred_element_type=jnp.float32)
                  + b2_ref[...])


def kernel(x, edge_index, Wl, bl, Wr, gn_w, gn_b, gn_ms, W1, b1, W2, b2):
    pad = E_PAD - E
    src3d = jnp.concatenate(
        [edge_index[0], jnp.zeros((pad,), jnp.int32)]).reshape(NW, NCHUNK, K)
    pad_dst = N + (jnp.arange(pad, dtype=jnp.int32) % (N_PAD - N))
    dst3d = jnp.concatenate([edge_index[1], pad_dst]).reshape(NW, NCHUNK, K)
    zeros = jnp.zeros((N_PAD, D), dtype=jnp.float32)
    partial = _sc_aggregate(x, src3d, dst3d, zeros)

    return pl.pallas_call(
        _tc_body,
        out_shape=jax.ShapeDtypeStruct((N, D), jnp.float32),
    )(partial, x, Wl.T, bl.reshape(1, D), Wr.T,
      gn_w.reshape(1, D), gn_b.reshape(1, D), gn_ms.reshape(1, D),
      W1.T, b1.reshape(1, D), W2.T, b2.reshape(1, D))


# speedup vs baseline: 2.4108x; 1.4210x over previous
import functools

import jax
import jax.numpy as jnp
from jax import lax
from jax.experimental import pallas as pl
from jax.experimental.pallas import tpu as pltpu
from jax.experimental.pallas import tpu_sc as plsc

N = 10000
D = 128
E = 320000
NC = 2
NS = 16
NW = NC * NS
K = 100
NCHUNK = 100
E_PAD = NW * NCHUNK * K
N_PAD = 10112
ROWS_PER_TILE = N_PAD // NS


def _sc_aggregate(x, src3d, dst3d, zeros):
    mesh = plsc.VectorSubcoreMesh(core_axis_name="c", subcore_axis_name="s")

    @functools.partial(
        pl.kernel,
        mesh=mesh,
        out_type=jax.ShapeDtypeStruct((NC, N_PAD, D), jnp.float32),
        scratch_types=[
            pltpu.VMEM((NCHUNK, K), jnp.int32),
            pltpu.VMEM((NCHUNK, K), jnp.int32),
            pltpu.VMEM((K, D), jnp.float32),
            pltpu.VMEM_SHARED((N_PAD, D), jnp.float32),
            pltpu.SemaphoreType.DMA,
        ],
    )
    def k(x_hbm, src_hbm, dst_hbm, zeros_hbm, out_hbm, src_v, dst_v, buf, acc, sem):
        c = lax.axis_index("c")
        s = lax.axis_index("s")
        wid = c * NS + s
        pltpu.sync_copy(src_hbm.at[wid], src_v)
        pltpu.sync_copy(dst_hbm.at[wid], dst_v)
        pltpu.sync_copy(
            zeros_hbm.at[pl.ds(s * ROWS_PER_TILE, ROWS_PER_TILE)],
            acc.at[pl.ds(s * ROWS_PER_TILE, ROWS_PER_TILE)],
        )
        plsc.subcore_barrier()

        def body(j, carry):
            pltpu.async_copy(x_hbm.at[src_v.at[j]], buf, sem).wait()
            pltpu.sync_copy(buf, acc.at[dst_v.at[j]], add=True)
            return carry

        lax.fori_loop(0, NCHUNK, body, 0)
        plsc.subcore_barrier()
        pltpu.sync_copy(
            acc.at[pl.ds(s * ROWS_PER_TILE, ROWS_PER_TILE)],
            out_hbm.at[c, pl.ds(s * ROWS_PER_TILE, ROWS_PER_TILE)],
        )

    return k(x, src3d, dst3d, zeros)


def _tc_body(p_ref, x_ref, wlt_ref, bl_ref, wrt_ref, gnw_ref, gnb_ref,
             gnms_ref, w1t_ref, b1_ref, w2t_ref, b2_ref, o_ref):
    agg = p_ref[0, :N, :] + p_ref[1, :N, :]
    h = (jnp.dot(agg, wlt_ref[...], preferred_element_type=jnp.float32)
         + bl_ref[...]
         + jnp.dot(x_ref[...], wrt_ref[...], preferred_element_type=jnp.float32))
    mean = jnp.mean(h, axis=0, keepdims=True)
    cen = h - gnms_ref[...] * mean
    var = jnp.mean(cen * cen, axis=0, keepdims=True)
    inv = lax.rsqrt(var + 1e-5)
    h = gnw_ref[...] * cen * inv + gnb_ref[...]
    h = jnp.maximum(h, 0.0)
    h = jnp.maximum(
        jnp.dot(h, w1t_ref[...], preferred_element_type=jnp.float32) + b1_ref[...],
        0.0)
    o_ref[...] = (jnp.dot(h, w2t_ref[...], preferred_element_type=jnp.float32)
                  + b2_ref[...])


def kernel(x, edge_index, Wl, bl, Wr, gn_w, gn_b, gn_ms, W1, b1, W2, b2):
    pad = E_PAD - E
    src3d = jnp.concatenate(
        [edge_index[0], jnp.zeros((pad,), jnp.int32)]).reshape(NW, NCHUNK, K)
    pad_dst = N + (jnp.arange(pad, dtype=jnp.int32) % (N_PAD - N))
    dst3d = jnp.concatenate([edge_index[1], pad_dst]).reshape(NW, NCHUNK, K)
    zeros = jnp.zeros((N_PAD, D), dtype=jnp.float32)
    partial = _sc_aggregate(x, src3d, dst3d, zeros)

    return pl.pallas_call(
        _tc_body,
        out_shape=jax.ShapeDtypeStruct((N, D), jnp.float32),
    )(partial, x, Wl.T, bl.reshape(1, D), Wr.T,
      gn_w.reshape(1, D), gn_b.reshape(1, D), gn_ms.reshape(1, D),
      W1.T, b1.reshape(1, D), W2.T, b2.reshape(1, D))


# speedup vs baseline: 2.5659x; 1.0643x over previous
import functools

import jax
import jax.numpy as jnp
from jax import lax
from jax.experimental import pallas as pl
from jax.experimental.pallas import tpu as pltpu
from jax.experimental.pallas import tpu_sc as plsc

N = 10000
D = 128
E = 320000
NC = 2
NS = 16
NW = NC * NS
K = 128
NCHUNK = 80
E_PAD = NW * NCHUNK * K
N_PAD = 10112
ROWS_PER_TILE = N_PAD // NS


def _sc_aggregate(x, src3d, dst3d, zeros):
    mesh = plsc.VectorSubcoreMesh(core_axis_name="c", subcore_axis_name="s")

    @functools.partial(
        pl.kernel,
        mesh=mesh,
        out_type=jax.ShapeDtypeStruct((NC, N_PAD, D), jnp.float32),
        scratch_types=[
            pltpu.VMEM((NCHUNK, K), jnp.int32),
            pltpu.VMEM((NCHUNK, K), jnp.int32),
            pltpu.VMEM((K, D), jnp.float32),
            pltpu.VMEM_SHARED((N_PAD, D), jnp.float32),
            pltpu.SemaphoreType.DMA,
        ],
    )
    def k(x_hbm, src_hbm, dst_hbm, zeros_hbm, out_hbm, src_v, dst_v, buf, acc, sem):
        c = lax.axis_index("c")
        s = lax.axis_index("s")
        wid = c * NS + s
        pltpu.sync_copy(src_hbm.at[wid], src_v)
        pltpu.sync_copy(dst_hbm.at[wid], dst_v)
        pltpu.sync_copy(
            zeros_hbm.at[pl.ds(s * ROWS_PER_TILE, ROWS_PER_TILE)],
            acc.at[pl.ds(s * ROWS_PER_TILE, ROWS_PER_TILE)],
        )
        plsc.subcore_barrier()

        def body(j, carry):
            pltpu.async_copy(x_hbm.at[src_v.at[j]], buf, sem).wait()
            pltpu.sync_copy(buf, acc.at[dst_v.at[j]], add=True)
            return carry

        lax.fori_loop(0, NCHUNK, body, 0)
        plsc.subcore_barrier()
        pltpu.sync_copy(
            acc.at[pl.ds(s * ROWS_PER_TILE, ROWS_PER_TILE)],
            out_hbm.at[c, pl.ds(s * ROWS_PER_TILE, ROWS_PER_TILE)],
        )

    return k(x, src3d, dst3d, zeros)


def _tc_body(p_ref, x_ref, wlt_ref, bl_ref, wrt_ref, gnw_ref, gnb_ref,
             gnms_ref, w1t_ref, b1_ref, w2t_ref, b2_ref, o_ref):
    agg = p_ref[0, :N, :] + p_ref[1, :N, :]
    h = (jnp.dot(agg, wlt_ref[...], preferred_element_type=jnp.float32)
         + bl_ref[...]
         + jnp.dot(x_ref[...], wrt_ref[...], preferred_element_type=jnp.float32))
    mean = jnp.mean(h, axis=0, keepdims=True)
    cen = h - gnms_ref[...] * mean
    var = jnp.mean(cen * cen, axis=0, keepdims=True)
    inv = lax.rsqrt(var + 1e-5)
    h = gnw_ref[...] * cen * inv + gnb_ref[...]
    h = jnp.maximum(h, 0.0)
    h = jnp.maximum(
        jnp.dot(h, w1t_ref[...], preferred_element_type=jnp.float32) + b1_ref[...],
        0.0)
    o_ref[...] = (jnp.dot(h, w2t_ref[...], preferred_element_type=jnp.float32)
                  + b2_ref[...])


def kernel(x, edge_index, Wl, bl, Wr, gn_w, gn_b, gn_ms, W1, b1, W2, b2):
    pad = E_PAD - E
    pad_src = jnp.arange(pad, dtype=jnp.int32) % N
    src3d = jnp.concatenate([edge_index[0], pad_src]).reshape(NW, NCHUNK, K)
    pad_dst = N + (jnp.arange(pad, dtype=jnp.int32) % (N_PAD - N))
    dst3d = jnp.concatenate([edge_index[1], pad_dst]).reshape(NW, NCHUNK, K)
    zeros = jnp.zeros((N_PAD, D), dtype=jnp.float32)
    partial = _sc_aggregate(x, src3d, dst3d, zeros)

    return pl.pallas_call(
        _tc_body,
        out_shape=jax.ShapeDtypeStruct((N, D), jnp.float32),
    )(partial, x, Wl.T, bl.reshape(1, D), Wr.T,
      gn_w.reshape(1, D), gn_b.reshape(1, D), gn_ms.reshape(1, D),
      W1.T, b1.reshape(1, D), W2.T, b2.reshape(1, D))
